# baseline (speedup 1.0000x reference)
import functools

import jax
import jax.numpy as jnp
from jax import lax
from jax.experimental import pallas as pl
from jax.experimental.pallas import tpu as pltpu
from jax.experimental.pallas import tpu_sc as plsc

NC = 2
NS = 16
NW = NC * NS
CHUNK = 128
HW = 8


KH = 8
KB = 8


def _hist_call(n_pad, n_chunks_per_tile):
  assert n_chunks_per_tile % KH == 0
  rpt = n_pad // NS
  mesh = plsc.VectorSubcoreMesh(core_axis_name="c", subcore_axis_name="s")

  @functools.partial(
      pl.kernel,
      mesh=mesh,
      compiler_params=pltpu.CompilerParams(use_tc_tiling_on_sc=False),
      out_type=jax.ShapeDtypeStruct((NC, 2 * n_pad, HW), jnp.float32),
      scratch_types=[
          pltpu.VMEM((n_chunks_per_tile, CHUNK), jnp.int32),
          pltpu.VMEM((n_chunks_per_tile, CHUNK), jnp.int32),
          pltpu.VMEM((CHUNK, HW), jnp.float32),
          pltpu.VMEM_SHARED((n_pad, HW), jnp.float32),
          pltpu.VMEM_SHARED((n_pad, HW), jnp.float32),
          pltpu.SemaphoreType.DMA,
      ],
  )
  def k(src_hbm, dst_hbm, zeros_hbm, ones_hbm, out_hbm,
        si_v, di_v, ones_v, hist_s, hist_d, sem):
    cid = lax.axis_index("c")
    sid = lax.axis_index("s")
    wid = sid * NC + cid
    pltpu.sync_copy(zeros_hbm, hist_s.at[pl.ds(sid * rpt, rpt)])
    pltpu.sync_copy(zeros_hbm, hist_d.at[pl.ds(sid * rpt, rpt)])
    pltpu.sync_copy(ones_hbm, ones_v)
    pltpu.sync_copy(src_hbm.at[wid], si_v)
    pltpu.sync_copy(dst_hbm.at[wid], di_v)
    plsc.subcore_barrier()

    def step(i, carry):
      g = i * KH
      for b in range(KH):
        pltpu.async_copy(ones_v, hist_s.at[si_v.at[g + b]], sem, add=True)
        pltpu.async_copy(ones_v, hist_d.at[di_v.at[g + b]], sem, add=True)
      for b in range(2 * KH):
        pltpu.make_async_copy(ones_v, hist_s.at[si_v.at[0]], sem).wait()
      return carry

    lax.fori_loop(0, n_chunks_per_tile // KH, step, 0)
    plsc.subcore_barrier()
    for hist, off in ((hist_s, 0), (hist_d, n_pad)):
      pltpu.sync_copy(hist.at[pl.ds(sid * rpt, rpt)],
                      out_hbm.at[cid, pl.ds(off + sid * rpt, rpt)])

  return k


def _mp_call(n_rows, d_pass, n_passes, n_chunks_per_tile):
  assert n_chunks_per_tile % KB == 0
  rpt = n_rows // NS
  mesh = plsc.VectorSubcoreMesh(core_axis_name="c", subcore_axis_name="s")
  n_outer = n_chunks_per_tile // KB

  @functools.partial(
      pl.kernel,
      mesh=mesh,
      compiler_params=pltpu.CompilerParams(use_tc_tiling_on_sc=False),
      out_type=jax.ShapeDtypeStruct((n_passes, NC, n_rows, d_pass),
                                    jnp.float32),
      scratch_types=(
          [pltpu.VMEM((n_chunks_per_tile, CHUNK), jnp.int32)] * 2
          + [pltpu.VMEM((CHUNK, d_pass), jnp.float32)] * KB
          + [pltpu.VMEM_SHARED((n_rows, d_pass), jnp.float32)] * 2
          + [pltpu.SemaphoreType.DMA] * (2 * KB)
      ),
  )
  def k(table_hbm, src_hbm, dst_hbm, zeros_hbm, out_hbm, *refs):
    si_v, di_v = refs[0], refs[1]
    msg = refs[2:2 + KB]
    agg_sh = refs[2 + KB]
    tab_sh = refs[3 + KB]
    sem_g = refs[4 + KB:4 + 2 * KB]
    sem_s = refs[4 + 2 * KB:4 + 3 * KB]
    cid = lax.axis_index("c")
    sid = lax.axis_index("s")
    wid = sid * NC + cid
    pltpu.sync_copy(src_hbm.at[wid], si_v)
    pltpu.sync_copy(dst_hbm.at[wid], di_v)

    for p in range(n_passes):
      pltpu.sync_copy(zeros_hbm, agg_sh.at[pl.ds(sid * rpt, rpt)])
      pltpu.sync_copy(table_hbm.at[p, pl.ds(sid * rpt, rpt)],
                      tab_sh.at[pl.ds(sid * rpt, rpt)])
      plsc.subcore_barrier()

      for b in range(KB):
        pltpu.async_copy(tab_sh.at[si_v.at[b]], msg[b], sem_g[b])

      def step(i, carry):
        g = i * KB
        for b in range(KB):
          pltpu.make_async_copy(tab_sh.at[si_v.at[0]], msg[b],
                                sem_g[b]).wait()
          pltpu.async_copy(msg[b], agg_sh.at[di_v.at[g + b]], sem_s[b],
                           add=True)
        for b in range(KB):
          pltpu.make_async_copy(msg[b], agg_sh.at[di_v.at[0]],
                                sem_s[b]).wait()
          c = g + b + KB

          @pl.when(c < n_chunks_per_tile)
          def _():
            pltpu.async_copy(tab_sh.at[si_v.at[c]], msg[b], sem_g[b])

        return carry

      lax.fori_loop(0, n_outer, step, 0)
      plsc.subcore_barrier()
      pltpu.sync_copy(agg_sh.at[pl.ds(sid * rpt, rpt)],
                      out_hbm.at[p, cid, pl.ds(sid * rpt, rpt)])

  return k


def _tc_mm0(x_pad, w1, n_pad):
  h = w1.shape[1]

  def body(x_ref, w_ref, h_ref):
    h_ref[...] = jnp.dot(x_ref[...], w_ref[...],
                         preferred_element_type=jnp.float32)

  return pl.pallas_call(
      body,
      out_shape=jax.ShapeDtypeStruct((n_pad, h), jnp.float32),
  )(x_pad, w1)


def _norms(hist_ref, n_pad):
  deg = hist_ref[0] + hist_ref[1]
  degc = deg[:, 0:1]
  norm = jnp.where(degc > 0, lax.rsqrt(degc), 0.0)
  return norm[0:n_pad], norm[n_pad:2 * n_pad]


def _tc_a(h1, histp, n_pad):
  h = h1.shape[1]

  def body(h1_ref, hist_ref, h_ref):
    ns, _ = _norms(hist_ref, n_pad)
    hh = h1_ref[...] * ns
    h_ref[0] = hh[:, :h // 2]
    h_ref[1] = hh[:, h // 2:]

  return pl.pallas_call(
      body,
      out_shape=jax.ShapeDtypeStruct((2, n_pad, h // 2), jnp.float32),
  )(h1, histp)


def _tc_b(p1, histp, b1, w2, n_pad):
  c = w2.shape[1]

  def body(p_ref, hist_ref, b_ref, w_ref, o_ref):
    ns, nd = _norms(hist_ref, n_pad)
    agg = jnp.concatenate(
        [p_ref[0, 0] + p_ref[0, 1], p_ref[1, 0] + p_ref[1, 1]], axis=1)
    hh = jnp.maximum(agg * nd + b_ref[...], 0.0)
    o_ref[0] = jnp.dot(hh, w_ref[...],
                       preferred_element_type=jnp.float32) * ns

  return pl.pallas_call(
      body,
      out_shape=jax.ShapeDtypeStruct((1, n_pad, c), jnp.float32),
  )(p1, histp, b1, w2)


def _tc_c(p2, histp, b2, n_pad, n):
  c = b2.shape[1]

  def body(p_ref, hist_ref, b_ref, o_ref):
    _, nd = _norms(hist_ref, n_pad)
    agg = p_ref[0, 0] + p_ref[0, 1]
    o_ref[...] = (agg * nd + b_ref[...])[:n]

  return pl.pallas_call(
      body,
      out_shape=jax.ShapeDtypeStruct((n, c), jnp.float32),
  )(p2, histp, b2)


@jax.jit
def kernel(in_feat, edge_index, W1, b1, W2, b2):
  n, d_in = in_feat.shape
  e = edge_index.shape[1]
  h = W1.shape[1]
  c = W2.shape[1]
  n_pad = ((n + 1023) // 1024) * 1024

  cpt = -(-e // (NW * CHUNK))
  cpt = -(-cpt // KB) * KB
  e_pad = NW * CHUNK * cpt

  src = edge_index[0]
  dst = edge_index[1]
  src_p = jnp.pad(src, (0, e_pad - e), constant_values=n)
  dst_p = jnp.pad(dst, (0, e_pad - e), constant_values=n)
  src3 = src_p.reshape(NW, cpt, CHUNK)
  dst3 = dst_p.reshape(NW, cpt, CHUNK)

  zeros_hist = jnp.zeros((n_pad // NS, HW), jnp.float32)
  ones_rows = jnp.ones((CHUNK, HW), jnp.float32)
  histp = _hist_call(n_pad, cpt)(src3, dst3, zeros_hist, ones_rows)

  x_pad = jnp.pad(in_feat, ((0, n_pad - n), (0, 0)))
  h1 = _tc_mm0(x_pad, W1, n_pad)
  h1n = _tc_a(h1, histp, n_pad)

  zeros_h = jnp.zeros((n_pad // NS, h // 2), jnp.float32)
  p1 = _mp_call(n_pad, h // 2, 2, cpt)(h1n, src3, dst3, zeros_h)

  h2n = _tc_b(p1, histp, b1.reshape(1, h), W2, n_pad)

  zeros_c = jnp.zeros((n_pad // NS, c), jnp.float32)
  p2 = _mp_call(n_pad, c, 1, cpt)(h2n, src3, dst3, zeros_c)

  return _tc_c(p2, histp, b2.reshape(1, c), n_pad, n)


# speedup vs baseline: 1.0249x; 1.0249x over previous
import functools

import jax
import jax.numpy as jnp
from jax import lax
from jax.experimental import pallas as pl
from jax.experimental.pallas import tpu as pltpu
from jax.experimental.pallas import tpu_sc as plsc

NC = 2
NS = 16
NW = NC * NS
CHUNK = 128
HW = 8


KH = 8
KB = 8


def _hist_call(n_pad, n_chunks_per_tile):
  assert n_chunks_per_tile % KH == 0
  rpt = n_pad // NS
  mesh = plsc.VectorSubcoreMesh(core_axis_name="c", subcore_axis_name="s")

  @functools.partial(
      pl.kernel,
      mesh=mesh,
      compiler_params=pltpu.CompilerParams(use_tc_tiling_on_sc=False),
      out_type=jax.ShapeDtypeStruct((NC, 2 * n_pad, HW), jnp.float32),
      scratch_types=[
          pltpu.VMEM((n_chunks_per_tile, CHUNK), jnp.int32),
          pltpu.VMEM((n_chunks_per_tile, CHUNK), jnp.int32),
          pltpu.VMEM((CHUNK, HW), jnp.float32),
          pltpu.VMEM_SHARED((n_pad, HW), jnp.float32),
          pltpu.VMEM_SHARED((n_pad, HW), jnp.float32),
          pltpu.SemaphoreType.DMA,
      ],
  )
  def k(src_hbm, dst_hbm, zeros_hbm, ones_hbm, out_hbm,
        si_v, di_v, ones_v, hist_s, hist_d, sem):
    cid = lax.axis_index("c")
    sid = lax.axis_index("s")
    wid = sid * NC + cid
    pltpu.sync_copy(zeros_hbm, hist_s.at[pl.ds(sid * rpt, rpt)])
    pltpu.sync_copy(zeros_hbm, hist_d.at[pl.ds(sid * rpt, rpt)])
    pltpu.sync_copy(ones_hbm, ones_v)
    pltpu.sync_copy(src_hbm.at[wid], si_v)
    pltpu.sync_copy(dst_hbm.at[wid], di_v)
    plsc.subcore_barrier()

    def step(i, carry):
      g = i * KH
      for b in range(KH):
        pltpu.async_copy(ones_v, hist_s.at[si_v.at[g + b]], sem, add=True)
        pltpu.async_copy(ones_v, hist_d.at[di_v.at[g + b]], sem, add=True)
      for b in range(2 * KH):
        pltpu.make_async_copy(ones_v, hist_s.at[si_v.at[0]], sem).wait()
      return carry

    lax.fori_loop(0, n_chunks_per_tile // KH, step, 0)
    plsc.subcore_barrier()
    for hist, off in ((hist_s, 0), (hist_d, n_pad)):
      pltpu.sync_copy(hist.at[pl.ds(sid * rpt, rpt)],
                      out_hbm.at[cid, pl.ds(off + sid * rpt, rpt)])

  return k


def _mp_call(n_rows, d_pass, n_passes, n_chunks_per_tile):
  assert n_chunks_per_tile % KB == 0
  rpt = n_rows // NS
  mesh = plsc.VectorSubcoreMesh(core_axis_name="c", subcore_axis_name="s")
  n_outer = n_chunks_per_tile // KB

  @functools.partial(
      pl.kernel,
      mesh=mesh,
      compiler_params=pltpu.CompilerParams(use_tc_tiling_on_sc=False),
      out_type=jax.ShapeDtypeStruct((n_passes, NC, n_rows, d_pass),
                                    jnp.float32),
      scratch_types=(
          [pltpu.VMEM((n_chunks_per_tile, CHUNK), jnp.int32)] * 2
          + [pltpu.VMEM((CHUNK, d_pass), jnp.float32)] * KB
          + [pltpu.VMEM_SHARED((n_rows, d_pass), jnp.float32)] * 2
          + [pltpu.SemaphoreType.DMA] * (2 * KB)
      ),
  )
  def k(table_hbm, src_hbm, dst_hbm, zeros_hbm, out_hbm, *refs):
    si_v, di_v = refs[0], refs[1]
    msg = refs[2:2 + KB]
    agg_sh = refs[2 + KB]
    tab_sh = refs[3 + KB]
    sem_g = refs[4 + KB:4 + 2 * KB]
    sem_s = refs[4 + 2 * KB:4 + 3 * KB]
    cid = lax.axis_index("c")
    sid = lax.axis_index("s")
    wid = sid * NC + cid
    pltpu.sync_copy(src_hbm.at[wid], si_v)
    pltpu.sync_copy(dst_hbm.at[wid], di_v)

    for p in range(n_passes):
      pltpu.sync_copy(zeros_hbm, agg_sh.at[pl.ds(sid * rpt, rpt)])
      pltpu.sync_copy(table_hbm.at[p, pl.ds(sid * rpt, rpt)],
                      tab_sh.at[pl.ds(sid * rpt, rpt)])
      plsc.subcore_barrier()

      for b in range(KB):
        pltpu.async_copy(tab_sh.at[si_v.at[b]], msg[b], sem_g[b])

      def step(i, carry):
        g = i * KB
        for b in range(KB):
          pltpu.make_async_copy(tab_sh.at[si_v.at[0]], msg[b],
                                sem_g[b]).wait()
          pltpu.async_copy(msg[b], agg_sh.at[di_v.at[g + b]], sem_s[b],
                           add=True)
        for b in range(KB):
          pltpu.make_async_copy(msg[b], agg_sh.at[di_v.at[0]],
                                sem_s[b]).wait()
          c = g + b + KB

          @pl.when(c < n_chunks_per_tile)
          def _():
            pltpu.async_copy(tab_sh.at[si_v.at[c]], msg[b], sem_g[b])

        return carry

      lax.fori_loop(0, n_outer, step, 0)
      plsc.subcore_barrier()
      pltpu.sync_copy(agg_sh.at[pl.ds(sid * rpt, rpt)],
                      out_hbm.at[p, cid, pl.ds(sid * rpt, rpt)])

  return k


def _tc_mm0(x_pad, w1, n_pad):
  h = w1.shape[1]

  def body(x_ref, w_ref, h_ref):
    h_ref[...] = jnp.dot(x_ref[...], w_ref[...],
                         preferred_element_type=jnp.float32)

  return pl.pallas_call(
      body,
      out_shape=jax.ShapeDtypeStruct((n_pad, h), jnp.float32),
  )(x_pad, w1)


def _norms(hist_ref, n_pad):
  deg = hist_ref[0] + hist_ref[1]
  degc = deg[:, 0:1]
  norm = jnp.where(degc > 0, lax.rsqrt(degc), 0.0)
  return norm[0:n_pad], norm[n_pad:2 * n_pad]


def _tc_a(h1, histp, n_pad):
  h = h1.shape[1]

  def body(h1_ref, hist_ref, h_ref, ns_ref, nd_ref):
    ns, nd = _norms(hist_ref, n_pad)
    hh = h1_ref[...] * ns
    h_ref[0] = hh[:, :h // 2]
    h_ref[1] = hh[:, h // 2:]
    ns_ref[...] = ns
    nd_ref[...] = nd

  return pl.pallas_call(
      body,
      out_shape=[
          jax.ShapeDtypeStruct((2, n_pad, h // 2), jnp.float32),
          jax.ShapeDtypeStruct((n_pad, 1), jnp.float32),
          jax.ShapeDtypeStruct((n_pad, 1), jnp.float32),
      ],
  )(h1, histp)


def _tc_b(p1, nd, ns, b1, w2, n_pad):
  c = w2.shape[1]

  def body(p_ref, nd_ref, ns_ref, b_ref, w_ref, o_ref):
    agg = jnp.concatenate(
        [p_ref[0, 0] + p_ref[0, 1], p_ref[1, 0] + p_ref[1, 1]], axis=1)
    hh = jnp.maximum(agg * nd_ref[...] + b_ref[...], 0.0)
    o_ref[0] = jnp.dot(hh, w_ref[...],
                       preferred_element_type=jnp.float32) * ns_ref[...]

  return pl.pallas_call(
      body,
      out_shape=jax.ShapeDtypeStruct((1, n_pad, c), jnp.float32),
  )(p1, nd, ns, b1, w2)


def _tc_c(p2, nd, b2, n):
  c = b2.shape[1]

  def body(p_ref, nd_ref, b_ref, o_ref):
    agg = p_ref[0, 0] + p_ref[0, 1]
    o_ref[...] = (agg * nd_ref[...] + b_ref[...])[:n]

  return pl.pallas_call(
      body,
      out_shape=jax.ShapeDtypeStruct((n, c), jnp.float32),
  )(p2, nd, b2)


@jax.jit
def kernel(in_feat, edge_index, W1, b1, W2, b2):
  n, d_in = in_feat.shape
  e = edge_index.shape[1]
  h = W1.shape[1]
  c = W2.shape[1]
  n_pad = ((n + 1023) // 1024) * 1024

  cpt = -(-e // (NW * CHUNK))
  cpt = -(-cpt // KB) * KB
  e_pad = NW * CHUNK * cpt

  src = edge_index[0]
  dst = edge_index[1]
  src_p = jnp.pad(src, (0, e_pad - e), constant_values=n)
  dst_p = jnp.pad(dst, (0, e_pad - e), constant_values=n)
  src3 = src_p.reshape(NW, cpt, CHUNK)
  dst3 = dst_p.reshape(NW, cpt, CHUNK)

  zeros_hist = jnp.zeros((n_pad // NS, HW), jnp.float32)
  ones_rows = jnp.ones((CHUNK, HW), jnp.float32)
  histp = _hist_call(n_pad, cpt)(src3, dst3, zeros_hist, ones_rows)

  x_pad = jnp.pad(in_feat, ((0, n_pad - n), (0, 0)))
  h1 = _tc_mm0(x_pad, W1, n_pad)
  h1n, ns, nd = _tc_a(h1, histp, n_pad)

  zeros_h = jnp.zeros((n_pad // NS, h // 2), jnp.float32)
  p1 = _mp_call(n_pad, h // 2, 2, cpt)(h1n, src3, dst3, zeros_h)

  h2n = _tc_b(p1, nd, ns, b1.reshape(1, h), W2, n_pad)

  zeros_c = jnp.zeros((n_pad // NS, c), jnp.float32)
  p2 = _mp_call(n_pad, c, 1, cpt)(h2n, src3, dst3, zeros_c)

  return _tc_c(p2, nd, b2.reshape(1, c), n)


# speedup vs baseline: 1.0464x; 1.0210x over previous
import functools

import jax
import jax.numpy as jnp
from jax import lax
from jax.experimental import pallas as pl
from jax.experimental.pallas import tpu as pltpu
from jax.experimental.pallas import tpu_sc as plsc

NC = 2
NS = 16
NW = NC * NS
CHUNK = 128
HW = 8


KH = 8
KB = 8


def _hist_call(n_pad, n_chunks_per_tile):
  assert n_chunks_per_tile % KH == 0
  rpt = n_pad // NS
  mesh = plsc.VectorSubcoreMesh(core_axis_name="c", subcore_axis_name="s")

  @functools.partial(
      pl.kernel,
      mesh=mesh,
      compiler_params=pltpu.CompilerParams(use_tc_tiling_on_sc=False),
      out_type=jax.ShapeDtypeStruct((NC, 2 * n_pad, HW), jnp.float32),
      scratch_types=[
          pltpu.VMEM((n_chunks_per_tile, CHUNK), jnp.int32),
          pltpu.VMEM((n_chunks_per_tile, CHUNK), jnp.int32),
          pltpu.VMEM((CHUNK, HW), jnp.float32),
          pltpu.VMEM_SHARED((n_pad, HW), jnp.float32),
          pltpu.VMEM_SHARED((n_pad, HW), jnp.float32),
          pltpu.SemaphoreType.DMA,
      ],
  )
  def k(src_hbm, dst_hbm, zeros_hbm, ones_hbm, out_hbm,
        si_v, di_v, ones_v, hist_s, hist_d, sem):
    cid = lax.axis_index("c")
    sid = lax.axis_index("s")
    wid = sid * NC + cid
    stage = [
        pltpu.async_copy(zeros_hbm, hist_s.at[pl.ds(sid * rpt, rpt)], sem),
        pltpu.async_copy(zeros_hbm, hist_d.at[pl.ds(sid * rpt, rpt)], sem),
        pltpu.async_copy(ones_hbm, ones_v, sem),
        pltpu.async_copy(src_hbm.at[wid], si_v, sem),
        pltpu.async_copy(dst_hbm.at[wid], di_v, sem),
    ]
    for d in stage:
      d.wait()
    plsc.subcore_barrier()

    def step(i, carry):
      g = i * KH
      for b in range(KH):
        pltpu.async_copy(ones_v, hist_s.at[si_v.at[g + b]], sem, add=True)
        pltpu.async_copy(ones_v, hist_d.at[di_v.at[g + b]], sem, add=True)
      for b in range(2 * KH):
        pltpu.make_async_copy(ones_v, hist_s.at[si_v.at[0]], sem).wait()
      return carry

    lax.fori_loop(0, n_chunks_per_tile // KH, step, 0)
    plsc.subcore_barrier()
    for hist, off in ((hist_s, 0), (hist_d, n_pad)):
      pltpu.sync_copy(hist.at[pl.ds(sid * rpt, rpt)],
                      out_hbm.at[cid, pl.ds(off + sid * rpt, rpt)])

  return k


def _mp_call(n_rows, d_pass, n_passes, n_chunks_per_tile):
  assert n_chunks_per_tile % KB == 0
  rpt = n_rows // NS
  mesh = plsc.VectorSubcoreMesh(core_axis_name="c", subcore_axis_name="s")
  n_outer = n_chunks_per_tile // KB

  @functools.partial(
      pl.kernel,
      mesh=mesh,
      compiler_params=pltpu.CompilerParams(use_tc_tiling_on_sc=False),
      out_type=jax.ShapeDtypeStruct((n_passes, NC, n_rows, d_pass),
                                    jnp.float32),
      scratch_types=(
          [pltpu.VMEM((n_chunks_per_tile, CHUNK), jnp.int32)] * 2
          + [pltpu.VMEM((CHUNK, d_pass), jnp.float32)] * KB
          + [pltpu.VMEM_SHARED((n_rows, d_pass), jnp.float32)] * 2
          + [pltpu.SemaphoreType.DMA] * (2 * KB)
      ),
  )
  def k(table_hbm, src_hbm, dst_hbm, zeros_hbm, out_hbm, *refs):
    si_v, di_v = refs[0], refs[1]
    msg = refs[2:2 + KB]
    agg_sh = refs[2 + KB]
    tab_sh = refs[3 + KB]
    sem_g = refs[4 + KB:4 + 2 * KB]
    sem_s = refs[4 + 2 * KB:4 + 3 * KB]
    cid = lax.axis_index("c")
    sid = lax.axis_index("s")
    wid = sid * NC + cid

    for p in range(n_passes):
      stage = [
          pltpu.async_copy(zeros_hbm, agg_sh.at[pl.ds(sid * rpt, rpt)],
                           sem_s[0]),
          pltpu.async_copy(table_hbm.at[p, pl.ds(sid * rpt, rpt)],
                           tab_sh.at[pl.ds(sid * rpt, rpt)], sem_s[1]),
      ]
      if p == 0:
        stage.append(pltpu.async_copy(src_hbm.at[wid], si_v, sem_s[2]))
        stage.append(pltpu.async_copy(dst_hbm.at[wid], di_v, sem_s[3]))
      for d in stage:
        d.wait()
      plsc.subcore_barrier()

      for b in range(KB):
        pltpu.async_copy(tab_sh.at[si_v.at[b]], msg[b], sem_g[b])

      def step(i, carry):
        g = i * KB
        for b in range(KB):
          pltpu.make_async_copy(tab_sh.at[si_v.at[0]], msg[b],
                                sem_g[b]).wait()
          pltpu.async_copy(msg[b], agg_sh.at[di_v.at[g + b]], sem_s[b],
                           add=True)
        for b in range(KB):
          pltpu.make_async_copy(msg[b], agg_sh.at[di_v.at[0]],
                                sem_s[b]).wait()
          c = g + b + KB

          @pl.when(c < n_chunks_per_tile)
          def _():
            pltpu.async_copy(tab_sh.at[si_v.at[c]], msg[b], sem_g[b])

        return carry

      lax.fori_loop(0, n_outer, step, 0)
      plsc.subcore_barrier()
      pltpu.sync_copy(agg_sh.at[pl.ds(sid * rpt, rpt)],
                      out_hbm.at[p, cid, pl.ds(sid * rpt, rpt)])

  return k


def _tc_mm0(x_pad, w1, n_pad):
  h = w1.shape[1]

  def body(x_ref, w_ref, h_ref):
    h_ref[...] = jnp.dot(x_ref[...], w_ref[...],
                         preferred_element_type=jnp.float32)

  return pl.pallas_call(
      body,
      out_shape=jax.ShapeDtypeStruct((n_pad, h), jnp.float32),
  )(x_pad, w1)


def _norms(hist_ref, n_pad):
  deg = hist_ref[0] + hist_ref[1]
  degc = deg[:, 0:1]
  norm = jnp.where(degc > 0, lax.rsqrt(degc), 0.0)
  return norm[0:n_pad], norm[n_pad:2 * n_pad]


def _tc_a(h1, histp, n_pad):
  h = h1.shape[1]

  def body(h1_ref, hist_ref, h_ref, ns_ref, nd_ref):
    ns, nd = _norms(hist_ref, n_pad)
    hh = h1_ref[...] * ns
    h_ref[0] = hh[:, :h // 2]
    h_ref[1] = hh[:, h // 2:]
    ns_ref[...] = ns
    nd_ref[...] = nd

  return pl.pallas_call(
      body,
      out_shape=[
          jax.ShapeDtypeStruct((2, n_pad, h // 2), jnp.float32),
          jax.ShapeDtypeStruct((n_pad, 1), jnp.float32),
          jax.ShapeDtypeStruct((n_pad, 1), jnp.float32),
      ],
  )(h1, histp)


def _tc_b(p1, nd, ns, b1, w2, n_pad):
  c = w2.shape[1]

  def body(p_ref, nd_ref, ns_ref, b_ref, w_ref, o_ref):
    agg = jnp.concatenate(
        [p_ref[0, 0] + p_ref[0, 1], p_ref[1, 0] + p_ref[1, 1]], axis=1)
    hh = jnp.maximum(agg * nd_ref[...] + b_ref[...], 0.0)
    o_ref[0] = jnp.dot(hh, w_ref[...],
                       preferred_element_type=jnp.float32) * ns_ref[...]

  return pl.pallas_call(
      body,
      out_shape=jax.ShapeDtypeStruct((1, n_pad, c), jnp.float32),
  )(p1, nd, ns, b1, w2)


def _tc_c(p2, nd, b2, n):
  c = b2.shape[1]

  def body(p_ref, nd_ref, b_ref, o_ref):
    agg = p_ref[0, 0] + p_ref[0, 1]
    o_ref[...] = (agg * nd_ref[...] + b_ref[...])[:n]

  return pl.pallas_call(
      body,
      out_shape=jax.ShapeDtypeStruct((n, c), jnp.float32),
  )(p2, nd, b2)


@jax.jit
def kernel(in_feat, edge_index, W1, b1, W2, b2):
  n, d_in = in_feat.shape
  e = edge_index.shape[1]
  h = W1.shape[1]
  c = W2.shape[1]
  n_pad = ((n + 1023) // 1024) * 1024

  cpt = -(-e // (NW * CHUNK))
  cpt = -(-cpt // KB) * KB
  e_pad = NW * CHUNK * cpt

  src = edge_index[0]
  dst = edge_index[1]
  src_p = jnp.pad(src, (0, e_pad - e), constant_values=n)
  dst_p = jnp.pad(dst, (0, e_pad - e), constant_values=n)
  src3 = src_p.reshape(NW, cpt, CHUNK)
  dst3 = dst_p.reshape(NW, cpt, CHUNK)

  zeros_hist = jnp.zeros((n_pad // NS, HW), jnp.float32)
  ones_rows = jnp.ones((CHUNK, HW), jnp.float32)
  histp = _hist_call(n_pad, cpt)(src3, dst3, zeros_hist, ones_rows)

  x_pad = jnp.pad(in_feat, ((0, n_pad - n), (0, 0)))
  h1 = _tc_mm0(x_pad, W1, n_pad)
  h1n, ns, nd = _tc_a(h1, histp, n_pad)

  zeros_h = jnp.zeros((n_pad // NS, h // 2), jnp.float32)
  p1 = _mp_call(n_pad, h // 2, 2, cpt)(h1n, src3, dst3, zeros_h)

  h2n = _tc_b(p1, nd, ns, b1.reshape(1, h), W2, n_pad)

  zeros_c = jnp.zeros((n_pad // NS, c), jnp.float32)
  p2 = _mp_call(n_pad, c, 1, cpt)(h2n, src3, dst3, zeros_c)

  return _tc_c(p2, nd, b2.reshape(1, c), n)
